# baseline (speedup 1.0000x reference)
import functools

import jax
import jax.numpy as jnp
from jax import lax
from jax.experimental import pallas as pl
from jax.experimental.pallas import tpu as pltpu
from jax.experimental.pallas import tpu_sc as plsc

NC = 2
NS = 16
NW = NC * NS
CHUNK = 128


def _sc_aggregate(h, send3, rec3, agg_rows, rows_per_tile):
    n, d = h.shape
    st = send3.shape[2]
    ch = send3.shape[1] * st
    zblocks = [CHUNK] * (rows_per_tile // CHUNK)
    if rows_per_tile % CHUNK:
        zblocks.append(rows_per_tile % CHUNK)

    mesh = plsc.VectorSubcoreMesh(core_axis_name="c", subcore_axis_name="s")

    @functools.partial(
        pl.kernel,
        out_type=jax.ShapeDtypeStruct((NC, agg_rows, d), jnp.float32),
        mesh=mesh,
        scratch_types=[
            pltpu.VMEM((st, CHUNK), jnp.int32),
            pltpu.VMEM((st, CHUNK), jnp.int32),
            pltpu.VMEM((CHUNK, d), jnp.float32),
            pltpu.VMEM((CHUNK, d), jnp.float32),
            pltpu.VMEM_SHARED((agg_rows, d), jnp.float32),
            pltpu.SemaphoreType.DMA,
            pltpu.SemaphoreType.DMA,
        ],
    )
    def agg_kernel(h_hbm, send_hbm, rec_hbm, out_hbm,
                   send_v, rec_v, rows_a, rows_b, agg_sh, sem0, sem1):
        c = lax.axis_index("c")
        s = lax.axis_index("s")
        wid = c * NS + s
        base = s * rows_per_tile

        zv = jnp.zeros((16,), jnp.float32)

        def zrow(i, carry):
            for k in range(d // 16):
                rows_a[i, pl.ds(k * 16, 16)] = zv
            return carry

        lax.fori_loop(0, CHUNK, zrow, 0)
        off = 0
        for zb in zblocks:
            pltpu.sync_copy(rows_a.at[pl.ds(0, zb)],
                            agg_sh.at[pl.ds(base + off, zb)])
            off += zb
        plsc.subcore_barrier()

        half = st // 2
        for g in range(ch // st):
            pltpu.sync_copy(send_hbm.at[wid, g], send_v)
            pltpu.sync_copy(rec_hbm.at[wid, g], rec_v)
            pltpu.async_copy(h_hbm.at[send_v.at[0]], rows_a, sem0)

            def pair(p, carry):
                j0 = 2 * p
                j1 = j0 + 1
                pltpu.make_async_copy(
                    h_hbm.at[send_v.at[j0]], rows_a, sem0).wait()
                pltpu.async_copy(h_hbm.at[send_v.at[j1]], rows_b, sem1)
                pltpu.sync_copy(rows_a, agg_sh.at[rec_v.at[j0]], add=True)
                pltpu.make_async_copy(
                    h_hbm.at[send_v.at[j1]], rows_b, sem1).wait()

                @pl.when(p + 1 < half)
                def _():
                    pltpu.async_copy(
                        h_hbm.at[send_v.at[j0 + 2]], rows_a, sem0)

                pltpu.sync_copy(rows_b, agg_sh.at[rec_v.at[j1]], add=True)
                return carry

            lax.fori_loop(0, half, pair, 0)
        plsc.subcore_barrier()

        off = 0
        for zb in zblocks:
            sl = pl.ds(base + off, zb)
            rb = rows_a.at[pl.ds(0, zb)]
            pltpu.sync_copy(agg_sh.at[sl], rb)
            pltpu.sync_copy(rb, out_hbm.at[c, sl])
            off += zb

    return agg_kernel(h, send3, rec3)


def _tc_linear(x, parts, w, b, block_rows):
    n, d = x.shape
    grid = (n // block_rows,)

    if parts is None:
        def body(x_ref, w_ref, b_ref, o_ref):
            o_ref[...] = (
                jnp.dot(x_ref[...], w_ref[...],
                        preferred_element_type=jnp.float32) + b_ref[...]
            )

        in_specs = [
            pl.BlockSpec((block_rows, d), lambda i: (i, 0)),
            pl.BlockSpec((d, d), lambda i: (0, 0)),
            pl.BlockSpec((1, d), lambda i: (0, 0)),
        ]
        operands = (x, w, b.reshape(1, d))
    else:
        def body(x_ref, p_ref, w_ref, b_ref, o_ref):
            acc = x_ref[...] + p_ref[0] + p_ref[1]
            o_ref[...] = (
                jnp.dot(acc, w_ref[...],
                        preferred_element_type=jnp.float32) + b_ref[...]
            )

        in_specs = [
            pl.BlockSpec((block_rows, d), lambda i: (i, 0)),
            pl.BlockSpec((NC, block_rows, d), lambda i: (0, i, 0)),
            pl.BlockSpec((d, d), lambda i: (0, 0)),
            pl.BlockSpec((1, d), lambda i: (0, 0)),
        ]
        operands = (x, parts, w, b.reshape(1, d))

    return pl.pallas_call(
        body,
        grid=grid,
        in_specs=in_specs,
        out_specs=pl.BlockSpec((block_rows, d), lambda i: (i, 0)),
        out_shape=jax.ShapeDtypeStruct((n, d), jnp.float32),
    )(*operands)


def kernel(h, edge_index, W_embed, b_embed, Wl, bl):
    n, d = h.shape
    e = edge_index.shape[1]
    n_layers = Wl.shape[0]

    per_tile = -(-e // NW)
    ch = -(-per_tile // CHUNK)
    ch = -(-ch // 4) * 4
    e_pad = NW * ch * CHUNK
    agg_rows = -(-(n + 1) // (NS * 8)) * (NS * 8)
    rows_per_tile = agg_rows // NS

    send = edge_index[0].astype(jnp.int32)
    rec = edge_index[1].astype(jnp.int32)
    pad = e_pad - e
    pad_rec = n + jnp.arange(pad, dtype=jnp.int32) % (agg_rows - n)
    send3 = jnp.concatenate(
        [send, jnp.zeros((pad,), jnp.int32)]).reshape(NW, 2, ch // 2, CHUNK)
    rec3 = jnp.concatenate([rec, pad_rec]).reshape(NW, 2, ch // 2, CHUNK)

    block_rows = 1000

    h = _tc_linear(h, None, W_embed, b_embed, block_rows)
    for i in range(n_layers):
        parts = _sc_aggregate(h, send3, rec3, agg_rows, rows_per_tile)
        h = _tc_linear(h, parts, Wl[i], bl[i], block_rows)
    return h


# speedup vs baseline: 1.0001x; 1.0001x over previous
import functools

import jax
import jax.numpy as jnp
from jax import lax
from jax.experimental import pallas as pl
from jax.experimental.pallas import tpu as pltpu
from jax.experimental.pallas import tpu_sc as plsc

NC = 2
NS = 16
NW = NC * NS
CHUNK = 128


def _sc_aggregate(h, send3, rec3, agg_rows, rows_per_tile):
    n, d = h.shape
    st = send3.shape[2]
    ch = send3.shape[1] * st
    zblocks = [CHUNK] * (rows_per_tile // CHUNK)
    if rows_per_tile % CHUNK:
        zblocks.append(rows_per_tile % CHUNK)

    mesh = plsc.VectorSubcoreMesh(core_axis_name="c", subcore_axis_name="s")

    @functools.partial(
        pl.kernel,
        out_type=jax.ShapeDtypeStruct((NC, agg_rows, d), jnp.float32),
        mesh=mesh,
        scratch_types=[
            pltpu.VMEM((st, CHUNK), jnp.int32),
            pltpu.VMEM((st, CHUNK), jnp.int32),
            pltpu.VMEM((CHUNK, d), jnp.float32),
            pltpu.VMEM((CHUNK, d), jnp.float32),
            pltpu.VMEM_SHARED((agg_rows, d), jnp.float32),
            pltpu.SemaphoreType.DMA,
            pltpu.SemaphoreType.DMA,
        ],
    )
    def agg_kernel(h_hbm, send_hbm, rec_hbm, out_hbm,
                   send_v, rec_v, rows_a, rows_b, agg_sh, sem0, sem1):
        c = lax.axis_index("c")
        s = lax.axis_index("s")
        wid = c * NS + s
        base = s * rows_per_tile

        zv = jnp.zeros((16,), jnp.float32)

        def zrow(i, carry):
            for k in range(d // 16):
                rows_a[i, pl.ds(k * 16, 16)] = zv
            return carry

        lax.fori_loop(0, CHUNK, zrow, 0)
        off = 0
        for zb in zblocks:
            pltpu.sync_copy(rows_a.at[pl.ds(0, zb)],
                            agg_sh.at[pl.ds(base + off, zb)])
            off += zb
        plsc.subcore_barrier()

        half = st // 2

        def stage(g, carry0):
            pltpu.sync_copy(send_hbm.at[wid, g], send_v)
            pltpu.sync_copy(rec_hbm.at[wid, g], rec_v)
            pltpu.async_copy(h_hbm.at[send_v.at[0]], rows_a, sem0)

            def pair(p, carry):
                j0 = 2 * p
                j1 = j0 + 1
                pltpu.make_async_copy(
                    h_hbm.at[send_v.at[j0]], rows_a, sem0).wait()
                pltpu.async_copy(h_hbm.at[send_v.at[j1]], rows_b, sem1)
                pltpu.sync_copy(rows_a, agg_sh.at[rec_v.at[j0]], add=True)
                pltpu.make_async_copy(
                    h_hbm.at[send_v.at[j1]], rows_b, sem1).wait()

                @pl.when(p + 1 < half)
                def _():
                    pltpu.async_copy(
                        h_hbm.at[send_v.at[j0 + 2]], rows_a, sem0)

                pltpu.sync_copy(rows_b, agg_sh.at[rec_v.at[j1]], add=True)
                return carry

            lax.fori_loop(0, half, pair, 0)
            return carry0

        lax.fori_loop(0, ch // st, stage, 0)
        plsc.subcore_barrier()

        off = 0
        for zb in zblocks:
            sl = pl.ds(base + off, zb)
            rb = rows_a.at[pl.ds(0, zb)]
            pltpu.sync_copy(agg_sh.at[sl], rb)
            pltpu.sync_copy(rb, out_hbm.at[c, sl])
            off += zb

    return agg_kernel(h, send3, rec3)


def _tc_linear(x, parts, w, b, block_rows):
    n, d = x.shape
    grid = (n // block_rows,)

    if parts is None:
        def body(x_ref, w_ref, b_ref, o_ref):
            o_ref[...] = (
                jnp.dot(x_ref[...], w_ref[...],
                        preferred_element_type=jnp.float32) + b_ref[...]
            )

        in_specs = [
            pl.BlockSpec((block_rows, d), lambda i: (i, 0)),
            pl.BlockSpec((d, d), lambda i: (0, 0)),
            pl.BlockSpec((1, d), lambda i: (0, 0)),
        ]
        operands = (x, w, b.reshape(1, d))
    else:
        def body(x_ref, p_ref, w_ref, b_ref, o_ref):
            acc = x_ref[...] + p_ref[0] + p_ref[1]
            o_ref[...] = (
                jnp.dot(acc, w_ref[...],
                        preferred_element_type=jnp.float32) + b_ref[...]
            )

        in_specs = [
            pl.BlockSpec((block_rows, d), lambda i: (i, 0)),
            pl.BlockSpec((NC, block_rows, d), lambda i: (0, i, 0)),
            pl.BlockSpec((d, d), lambda i: (0, 0)),
            pl.BlockSpec((1, d), lambda i: (0, 0)),
        ]
        operands = (x, parts, w, b.reshape(1, d))

    return pl.pallas_call(
        body,
        grid=grid,
        in_specs=in_specs,
        out_specs=pl.BlockSpec((block_rows, d), lambda i: (i, 0)),
        out_shape=jax.ShapeDtypeStruct((n, d), jnp.float32),
    )(*operands)


def kernel(h, edge_index, W_embed, b_embed, Wl, bl):
    n, d = h.shape
    e = edge_index.shape[1]
    n_layers = Wl.shape[0]

    per_tile = -(-e // NW)
    ch = -(-per_tile // CHUNK)
    ch = -(-ch // 4) * 4
    e_pad = NW * ch * CHUNK
    agg_rows = -(-(n + 1) // (NS * 8)) * (NS * 8)
    rows_per_tile = agg_rows // NS

    send = edge_index[0].astype(jnp.int32)
    rec = edge_index[1].astype(jnp.int32)
    pad = e_pad - e
    pad_rec = n + jnp.arange(pad, dtype=jnp.int32) % (agg_rows - n)
    send3 = jnp.concatenate(
        [send, jnp.zeros((pad,), jnp.int32)]).reshape(NW, 2, ch // 2, CHUNK)
    rec3 = jnp.concatenate([rec, pad_rec]).reshape(NW, 2, ch // 2, CHUNK)

    block_rows = 1000

    h = _tc_linear(h, None, W_embed, b_embed, block_rows)
    for i in range(n_layers):
        parts = _sc_aggregate(h, send3, rec3, agg_rows, rows_per_tile)
        h = _tc_linear(h, parts, Wl[i], bl[i], block_rows)
    return h


# speedup vs baseline: 1.0068x; 1.0067x over previous
import functools

import jax
import jax.numpy as jnp
from jax import lax
from jax.experimental import pallas as pl
from jax.experimental.pallas import tpu as pltpu
from jax.experimental.pallas import tpu_sc as plsc

NC = 2
NS = 16
NW = NC * NS
CHUNK = 128


def _sc_aggregate(h, send3, rec3, agg_rows, rows_per_tile):
    n, d = h.shape
    st = send3.shape[2]
    ch = send3.shape[1] * st
    zblocks = [CHUNK] * (rows_per_tile // CHUNK)
    if rows_per_tile % CHUNK:
        zblocks.append(rows_per_tile % CHUNK)

    mesh = plsc.VectorSubcoreMesh(core_axis_name="c", subcore_axis_name="s")

    @functools.partial(
        pl.kernel,
        out_type=jax.ShapeDtypeStruct((NC, agg_rows, d), jnp.float32),
        mesh=mesh,
        scratch_types=[
            pltpu.VMEM((st, CHUNK), jnp.int32),
            pltpu.VMEM((st, CHUNK), jnp.int32),
            pltpu.VMEM((CHUNK, d), jnp.float32),
            pltpu.VMEM((CHUNK, d), jnp.float32),
            pltpu.VMEM_SHARED((agg_rows, d), jnp.float32),
            pltpu.SemaphoreType.DMA,
            pltpu.SemaphoreType.DMA,
            pltpu.SemaphoreType.DMA,
            pltpu.SemaphoreType.DMA,
        ],
    )
    def agg_kernel(h_hbm, send_hbm, rec_hbm, out_hbm,
                   send_v, rec_v, rows_a, rows_b, agg_sh,
                   sem_ga, sem_gb, sem_sa, sem_sb):
        c = lax.axis_index("c")
        s = lax.axis_index("s")
        wid = c * NS + s
        base = s * rows_per_tile

        zv = jnp.zeros((16,), jnp.float32)

        def zrow(i, carry):
            for k in range(d // 16):
                rows_a[i, pl.ds(k * 16, 16)] = zv
            return carry

        lax.fori_loop(0, CHUNK, zrow, 0)
        off = 0
        for zb in zblocks:
            pltpu.sync_copy(rows_a.at[pl.ds(0, zb)],
                            agg_sh.at[pl.ds(base + off, zb)])
            off += zb
        plsc.subcore_barrier()

        half = st // 2

        def stage(g, carry0):
            pltpu.sync_copy(send_hbm.at[wid, g], send_v)
            pltpu.sync_copy(rec_hbm.at[wid, g], rec_v)
            pltpu.async_copy(h_hbm.at[send_v.at[0]], rows_a, sem_ga)
            pltpu.async_copy(h_hbm.at[send_v.at[1]], rows_b, sem_gb)

            def pair(p, carry):
                j0 = 2 * p
                j1 = j0 + 1
                pltpu.make_async_copy(
                    h_hbm.at[send_v.at[j0]], rows_a, sem_ga).wait()
                pltpu.async_copy(rows_a, agg_sh.at[rec_v.at[j0]], sem_sa,
                                 add=True)
                pltpu.make_async_copy(
                    h_hbm.at[send_v.at[j1]], rows_b, sem_gb).wait()
                pltpu.async_copy(rows_b, agg_sh.at[rec_v.at[j1]], sem_sb,
                                 add=True)

                @pl.when(p + 1 < half)
                def _():
                    pltpu.make_async_copy(
                        rows_a, agg_sh.at[rec_v.at[j0]], sem_sa).wait()
                    pltpu.async_copy(
                        h_hbm.at[send_v.at[j0 + 2]], rows_a, sem_ga)
                    pltpu.make_async_copy(
                        rows_b, agg_sh.at[rec_v.at[j1]], sem_sb).wait()
                    pltpu.async_copy(
                        h_hbm.at[send_v.at[j1 + 2]], rows_b, sem_gb)

                return carry

            lax.fori_loop(0, half, pair, 0)
            pltpu.make_async_copy(
                rows_a, agg_sh.at[rec_v.at[st - 2]], sem_sa).wait()
            pltpu.make_async_copy(
                rows_b, agg_sh.at[rec_v.at[st - 1]], sem_sb).wait()
            return carry0

        lax.fori_loop(0, ch // st, stage, 0)
        plsc.subcore_barrier()

        off = 0
        for zb in zblocks:
            sl = pl.ds(base + off, zb)
            rb = rows_a.at[pl.ds(0, zb)]
            pltpu.sync_copy(agg_sh.at[sl], rb)
            pltpu.sync_copy(rb, out_hbm.at[c, sl])
            off += zb

    return agg_kernel(h, send3, rec3)


def _tc_linear(x, parts, w, b, block_rows):
    n, d = x.shape
    grid = (n // block_rows,)

    if parts is None:
        def body(x_ref, w_ref, b_ref, o_ref):
            o_ref[...] = (
                jnp.dot(x_ref[...], w_ref[...],
                        preferred_element_type=jnp.float32) + b_ref[...]
            )

        in_specs = [
            pl.BlockSpec((block_rows, d), lambda i: (i, 0)),
            pl.BlockSpec((d, d), lambda i: (0, 0)),
            pl.BlockSpec((1, d), lambda i: (0, 0)),
        ]
        operands = (x, w, b.reshape(1, d))
    else:
        def body(x_ref, p_ref, w_ref, b_ref, o_ref):
            acc = x_ref[...] + p_ref[0] + p_ref[1]
            o_ref[...] = (
                jnp.dot(acc, w_ref[...],
                        preferred_element_type=jnp.float32) + b_ref[...]
            )

        in_specs = [
            pl.BlockSpec((block_rows, d), lambda i: (i, 0)),
            pl.BlockSpec((NC, block_rows, d), lambda i: (0, i, 0)),
            pl.BlockSpec((d, d), lambda i: (0, 0)),
            pl.BlockSpec((1, d), lambda i: (0, 0)),
        ]
        operands = (x, parts, w, b.reshape(1, d))

    return pl.pallas_call(
        body,
        grid=grid,
        in_specs=in_specs,
        out_specs=pl.BlockSpec((block_rows, d), lambda i: (i, 0)),
        out_shape=jax.ShapeDtypeStruct((n, d), jnp.float32),
    )(*operands)


def kernel(h, edge_index, W_embed, b_embed, Wl, bl):
    n, d = h.shape
    e = edge_index.shape[1]
    n_layers = Wl.shape[0]

    per_tile = -(-e // NW)
    ch = -(-per_tile // CHUNK)
    ch = -(-ch // 4) * 4
    e_pad = NW * ch * CHUNK
    agg_rows = -(-(n + 1) // (NS * 8)) * (NS * 8)
    rows_per_tile = agg_rows // NS

    send = edge_index[0].astype(jnp.int32)
    rec = edge_index[1].astype(jnp.int32)
    pad = e_pad - e
    pad_rec = n + jnp.arange(pad, dtype=jnp.int32) % (agg_rows - n)
    send3 = jnp.concatenate(
        [send, jnp.zeros((pad,), jnp.int32)]).reshape(NW, 2, ch // 2, CHUNK)
    rec3 = jnp.concatenate([rec, pad_rec]).reshape(NW, 2, ch // 2, CHUNK)

    block_rows = 1000

    h = _tc_linear(h, None, W_embed, b_embed, block_rows)
    for i in range(n_layers):
        parts = _sc_aggregate(h, send3, rec3, agg_rows, rows_per_tile)
        h = _tc_linear(h, parts, Wl[i], bl[i], block_rows)
    return h
